# baseline (speedup 1.0000x reference)
import functools

import jax
import jax.numpy as jnp
from jax import lax
from jax.experimental import pallas as pl
from jax.experimental.pallas import tpu as pltpu
from jax.experimental.pallas import tpu_sc as plsc

N = 10000
D = 128
E = 320000

NC = 2
NS = 16
NW = NC * NS
CHUNK = 128
CPW = 80
EPAD = NW * CPW * CHUNK
NPAD = 10240
RPT = NPAD // NS

BN = 512
GRID = NPAD // BN

KS = 80
KF = 80
NKS0 = NS * KS
TOTCH = NW * CPW
DH = D // 2

_f32 = jnp.float32


def _mesh():
    return plsc.VectorSubcoreMesh(core_axis_name="c", subcore_axis_name="s")


def _deg_body(dst_hbm, zeros_hbm, ones_hbm, out_hbm, idx_v, ones_v, acc_sh, ssem):
    c = lax.axis_index("c")
    s = lax.axis_index("s")
    wid = c * NS + s
    r0 = s * RPT
    pltpu.sync_copy(zeros_hbm.at[pl.ds(r0, RPT)], acc_sh.at[pl.ds(r0, RPT)])
    pltpu.sync_copy(ones_hbm, ones_v)
    pltpu.sync_copy(dst_hbm.at[wid], idx_v)
    plsc.subcore_barrier()

    def fire(j, carry):
        pltpu.async_copy(ones_v, acc_sh.at[idx_v.at[j]], ssem, add=True)
        return carry

    lax.fori_loop(0, CPW, fire, 0)

    def drain(j, carry):
        pltpu.make_async_copy(ones_v, acc_sh.at[idx_v.at[0]], ssem).wait()
        return carry

    lax.fori_loop(0, CPW, drain, 0)
    plsc.subcore_barrier()
    pltpu.sync_copy(acc_sh.at[pl.ds(r0, RPT)], out_hbm.at[c, pl.ds(r0, RPT)])


def _sc_degree(dstp, zeros128, ones128):
    return pl.kernel(
        _deg_body,
        out_type=jax.ShapeDtypeStruct((NC, NPAD, D), _f32),
        mesh=_mesh(),
        scratch_types=[
            pltpu.VMEM((CPW, CHUNK), jnp.int32),
            pltpu.VMEM((CHUNK, D), _f32),
            pltpu.VMEM_SHARED((NPAD, D), _f32),
            pltpu.SemaphoreType.DMA,
        ],
    )(dstp, zeros128, ones128)


def _agg_body(table_hbm, sd_hbm, zeros_hbm, out_hbm,
              sd_r, rows_v, acc_sh, isem0, isem1, gsem0, gsem1):
    c = lax.axis_index("c")
    s = lax.axis_index("s")
    r0 = s * RPT
    base = jnp.where(c == 0, s * KS, NKS0 + s * KF)
    nch = jnp.where(c == 0, KS, KF)
    isems = (isem0, isem1)
    gsems = (gsem0, gsem1)
    pltpu.sync_copy(zeros_hbm.at[pl.ds(r0, RPT)], acc_sh.at[pl.ds(r0, RPT)])

    pltpu.async_copy(sd_hbm.at[base], sd_r.at[0], isems[0])
    pltpu.async_copy(sd_hbm.at[base + 1], sd_r.at[1], isems[1])
    plsc.subcore_barrier()
    pltpu.make_async_copy(sd_hbm.at[base], sd_r.at[0], isems[0]).wait()
    pltpu.async_copy(table_hbm.at[sd_r.at[0, 0]], rows_v.at[0], gsems[0])

    def group(g, carry):
        for b in range(2):
            j = 2 * g + b
            nb = 1 - b

            @pl.when(j + 1 < nch)
            def _fire_gather():
                pltpu.make_async_copy(
                    sd_hbm.at[base], sd_r.at[nb], isems[nb]).wait()
                pltpu.async_copy(
                    table_hbm.at[sd_r.at[nb, 0]], rows_v.at[nb], gsems[nb])

            pltpu.make_async_copy(
                table_hbm.at[sd_r.at[b, 0]], rows_v.at[b], gsems[b]).wait()
            pltpu.sync_copy(rows_v.at[b], acc_sh.at[sd_r.at[b, 1]], add=True)

            @pl.when(j + 2 < nch)
            def _fire_idx():
                pltpu.async_copy(
                    sd_hbm.at[base + (j + 2)], sd_r.at[b], isems[b])
        return carry

    lax.fori_loop(0, lax.div(nch, 2), group, 0)
    plsc.subcore_barrier()
    pltpu.sync_copy(acc_sh.at[pl.ds(r0, RPT)], out_hbm.at[c, pl.ds(r0, RPT)])


def _sc_aggregate(table, sd_flat, zeros128):
    return pl.kernel(
        _agg_body,
        out_type=jax.ShapeDtypeStruct((NC, NPAD, D), _f32),
        mesh=_mesh(),
        scratch_types=[
            pltpu.VMEM((2, 2, CHUNK), jnp.int32),
            pltpu.VMEM((2, CHUNK, D), _f32),
            pltpu.VMEM_SHARED((NPAD, D), _f32),
            pltpu.SemaphoreType.DMA,
            pltpu.SemaphoreType.DMA,
            pltpu.SemaphoreType.DMA,
            pltpu.SemaphoreType.DMA,
        ],
    )(table, sd_flat, zeros128)


def _mm_body(h_ref, w_ref, o_ref):
    o_ref[...] = jnp.dot(h_ref[...], w_ref[...], preferred_element_type=_f32)


def _tc_matmul(h, w):
    return pl.pallas_call(
        _mm_body,
        grid=(GRID,),
        in_specs=[
            pl.BlockSpec((BN, D), lambda i: (i, 0)),
            pl.BlockSpec((D, D), lambda i: (0, 0)),
        ],
        out_specs=pl.BlockSpec((BN, D), lambda i: (i, 0)),
        out_shape=jax.ShapeDtypeStruct((NPAD, D), _f32),
    )(h, w)


def _scale1_body(degp_ref, u_ref, mm_ref, dinv_ref):
    deg = degp_ref[0] + degp_ref[1] + 1.0
    dinv = jnp.where(deg > 0.0, lax.rsqrt(deg), 0.0)
    dinv_ref[...] = dinv
    mm_ref[...] = u_ref[...] * dinv


def _tc_scale1(deg_p, u1):
    return pl.pallas_call(
        _scale1_body,
        grid=(GRID,),
        in_specs=[
            pl.BlockSpec((NC, BN, D), lambda i: (0, i, 0)),
            pl.BlockSpec((BN, D), lambda i: (i, 0)),
        ],
        out_specs=[
            pl.BlockSpec((BN, D), lambda i: (i, 0)),
            pl.BlockSpec((BN, D), lambda i: (i, 0)),
        ],
        out_shape=[
            jax.ShapeDtypeStruct((NPAD, D), _f32),
            jax.ShapeDtypeStruct((NPAD, D), _f32),
        ],
    )(deg_p, u1)


def _layer_body(has_skip, *refs):
    if has_skip:
        a_ref, mm_ref, dinv_ref, b_ref, w_ref, skip_ref, h_ref, mmn_ref = refs
    else:
        a_ref, mm_ref, dinv_ref, b_ref, w_ref, h_ref, mmn_ref = refs
    dinv = dinv_ref[...]
    t = (a_ref[0] + a_ref[1] + mm_ref[...]) * dinv + b_ref[...]
    h = jnp.maximum(t, 0.0)
    if has_skip:
        h = h + skip_ref[...]
    h_ref[...] = h
    mmn_ref[...] = jnp.dot(h, w_ref[...], preferred_element_type=_f32) * dinv


def _tc_layer(a_p, mm, dinv16, b_row, w_next, skip=None):
    has_skip = skip is not None
    in_specs = [
        pl.BlockSpec((NC, BN, D), lambda i: (0, i, 0)),
        pl.BlockSpec((BN, D), lambda i: (i, 0)),
        pl.BlockSpec((BN, D), lambda i: (i, 0)),
        pl.BlockSpec((1, D), lambda i: (0, 0)),
        pl.BlockSpec((D, D), lambda i: (0, 0)),
    ]
    args = [a_p, mm, dinv16, b_row, w_next]
    if has_skip:
        in_specs.append(pl.BlockSpec((BN, D), lambda i: (i, 0)))
        args.append(skip)
    return pl.pallas_call(
        functools.partial(_layer_body, has_skip),
        grid=(GRID,),
        in_specs=in_specs,
        out_specs=[
            pl.BlockSpec((BN, D), lambda i: (i, 0)),
            pl.BlockSpec((BN, D), lambda i: (i, 0)),
        ],
        out_shape=[
            jax.ShapeDtypeStruct((NPAD, D), _f32),
            jax.ShapeDtypeStruct((NPAD, D), _f32),
        ],
    )(*args)


def _final_body(a_ref, mm_ref, dinv_ref, b_ref, x_ref, o_ref):
    t = (a_ref[0] + a_ref[1] + mm_ref[...]) * dinv_ref[...] + b_ref[...]
    o_ref[...] = jnp.maximum(t, 0.0) + x_ref[...]


def _tc_final(a_p, mm, dinv16, b_row, xp):
    return pl.pallas_call(
        _final_body,
        grid=(GRID,),
        in_specs=[
            pl.BlockSpec((NC, BN, D), lambda i: (0, i, 0)),
            pl.BlockSpec((BN, D), lambda i: (i, 0)),
            pl.BlockSpec((BN, D), lambda i: (i, 0)),
            pl.BlockSpec((1, D), lambda i: (0, 0)),
            pl.BlockSpec((BN, D), lambda i: (i, 0)),
        ],
        out_specs=pl.BlockSpec((BN, D), lambda i: (i, 0)),
        out_shape=jax.ShapeDtypeStruct((NPAD, D), _f32),
    )(a_p, mm, dinv16, b_row, xp)


def kernel(x, edge_index, W1, b1, W2, b2, W3, b3, W4, b4):
    ei = edge_index.astype(jnp.int32)
    src_s, dst_s = lax.sort((ei[0], ei[1]), num_keys=1)
    pad = jnp.full((EPAD - E,), N, dtype=jnp.int32)
    srcc = jnp.concatenate([src_s, pad]).reshape(TOTCH, CHUNK)
    dstc = jnp.concatenate([dst_s, pad]).reshape(TOTCH, CHUNK)
    dstp = dstc.reshape(NW, CPW, CHUNK)
    sdp = jnp.stack([srcc, dstc], axis=1)

    xp = jnp.pad(x.astype(_f32), ((0, NPAD - N), (0, 0)))
    zeros128 = jnp.zeros((NPAD, D), _f32)
    ones128 = jnp.ones((CHUNK, D), _f32)
    b1r, b2r, b3r, b4r = (b.reshape(1, D) for b in (b1, b2, b3, b4))

    deg_p = _sc_degree(dstp, zeros128, ones128)
    u1 = _tc_matmul(xp, W1)
    mm1, dinv_b = _tc_scale1(deg_p, u1)

    a1 = _sc_aggregate(mm1, sdp, zeros128)
    h1, mm2 = _tc_layer(a1, mm1, dinv_b, b1r, W2)
    a2 = _sc_aggregate(mm2, sdp, zeros128)
    _, mm3 = _tc_layer(a2, mm2, dinv_b, b2r, W3)
    a3 = _sc_aggregate(mm3, sdp, zeros128)
    _, mm4 = _tc_layer(a3, mm3, dinv_b, b3r, W4, skip=h1)
    a4 = _sc_aggregate(mm4, sdp, zeros128)
    out = _tc_final(a4, mm4, dinv_b, b4r, xp)
    return out[:N]


# speedup vs baseline: 1.3379x; 1.3379x over previous
import functools

import jax
import jax.numpy as jnp
from jax import lax
from jax.experimental import pallas as pl
from jax.experimental.pallas import tpu as pltpu
from jax.experimental.pallas import tpu_sc as plsc

N = 10000
D = 128
E = 320000

NC = 2
NS = 16
NW = NC * NS
CHUNK = 128
CPW = 80
EPAD = NW * CPW * CHUNK
NPAD = 10240
RPT = NPAD // NS

BN = 512
GRID = NPAD // BN

KS = 110
KF = 50
NKS0 = NS * KS
TOTCH = NW * CPW
DH = D // 2

_f32 = jnp.float32


def _mesh():
    return plsc.VectorSubcoreMesh(core_axis_name="c", subcore_axis_name="s")


def _deg_body(dst_hbm, zeros_hbm, ones_hbm, out_hbm, idx_v, ones_v, acc_sh, ssem):
    c = lax.axis_index("c")
    s = lax.axis_index("s")
    wid = c * NS + s
    r0 = s * RPT
    pltpu.sync_copy(zeros_hbm.at[pl.ds(r0, RPT)], acc_sh.at[pl.ds(r0, RPT)])
    pltpu.sync_copy(ones_hbm, ones_v)
    pltpu.sync_copy(dst_hbm.at[wid], idx_v)
    plsc.subcore_barrier()

    def fire(j, carry):
        pltpu.async_copy(ones_v, acc_sh.at[idx_v.at[j]], ssem, add=True)
        return carry

    lax.fori_loop(0, CPW, fire, 0)

    def drain(j, carry):
        pltpu.make_async_copy(ones_v, acc_sh.at[idx_v.at[0]], ssem).wait()
        return carry

    lax.fori_loop(0, CPW, drain, 0)
    plsc.subcore_barrier()
    pltpu.sync_copy(acc_sh.at[pl.ds(r0, RPT)], out_hbm.at[c, pl.ds(r0, RPT)])


def _sc_degree(dstp, zeros128, ones128):
    return pl.kernel(
        _deg_body,
        out_type=jax.ShapeDtypeStruct((NC, NPAD, D), _f32),
        mesh=_mesh(),
        scratch_types=[
            pltpu.VMEM((CPW, CHUNK), jnp.int32),
            pltpu.VMEM((CHUNK, D), _f32),
            pltpu.VMEM_SHARED((NPAD, D), _f32),
            pltpu.SemaphoreType.DMA,
        ],
    )(dstp, zeros128, ones128)


def _agg_body(table_hbm, sd_hbm, zeros_hbm, out_hbm,
              sd_r, rows_v, acc_sh, isem0, isem1, gsem0, gsem1):
    c = lax.axis_index("c")
    s = lax.axis_index("s")
    r0 = s * RPT
    base = jnp.where(c == 0, s * KS, NKS0 + s * KF)
    nch = jnp.where(c == 0, KS, KF)
    isems = (isem0, isem1)
    gsems = (gsem0, gsem1)
    pltpu.sync_copy(zeros_hbm.at[pl.ds(r0, RPT)], acc_sh.at[pl.ds(r0, RPT)])

    pltpu.async_copy(sd_hbm.at[base], sd_r.at[0], isems[0])
    pltpu.async_copy(sd_hbm.at[base + 1], sd_r.at[1], isems[1])
    plsc.subcore_barrier()
    pltpu.make_async_copy(sd_hbm.at[base], sd_r.at[0], isems[0]).wait()
    pltpu.async_copy(table_hbm.at[sd_r.at[0, 0]], rows_v.at[0], gsems[0])

    def group(g, carry):
        for b in range(2):
            j = 2 * g + b
            nb = 1 - b

            @pl.when(j + 1 < nch)
            def _fire_gather():
                pltpu.make_async_copy(
                    sd_hbm.at[base], sd_r.at[nb], isems[nb]).wait()
                pltpu.async_copy(
                    table_hbm.at[sd_r.at[nb, 0]], rows_v.at[nb], gsems[nb])

            pltpu.make_async_copy(
                table_hbm.at[sd_r.at[b, 0]], rows_v.at[b], gsems[b]).wait()
            pltpu.sync_copy(rows_v.at[b], acc_sh.at[sd_r.at[b, 1]], add=True)

            @pl.when(j + 2 < nch)
            def _fire_idx():
                pltpu.async_copy(
                    sd_hbm.at[base + (j + 2)], sd_r.at[b], isems[b])
        return carry

    lax.fori_loop(0, lax.div(nch, 2), group, 0)
    plsc.subcore_barrier()
    pltpu.sync_copy(acc_sh.at[pl.ds(r0, RPT)], out_hbm.at[c, pl.ds(r0, RPT)])


def _sc_aggregate(table, sd_flat, zeros128):
    return pl.kernel(
        _agg_body,
        out_type=jax.ShapeDtypeStruct((NC, NPAD, D), _f32),
        mesh=_mesh(),
        scratch_types=[
            pltpu.VMEM((2, 2, CHUNK), jnp.int32),
            pltpu.VMEM((2, CHUNK, D), _f32),
            pltpu.VMEM_SHARED((NPAD, D), _f32),
            pltpu.SemaphoreType.DMA,
            pltpu.SemaphoreType.DMA,
            pltpu.SemaphoreType.DMA,
            pltpu.SemaphoreType.DMA,
        ],
    )(table, sd_flat, zeros128)


def _mm_body(h_ref, w_ref, o_ref):
    o_ref[...] = jnp.dot(h_ref[...], w_ref[...], preferred_element_type=_f32)


def _tc_matmul(h, w):
    return pl.pallas_call(
        _mm_body,
        grid=(GRID,),
        in_specs=[
            pl.BlockSpec((BN, D), lambda i: (i, 0)),
            pl.BlockSpec((D, D), lambda i: (0, 0)),
        ],
        out_specs=pl.BlockSpec((BN, D), lambda i: (i, 0)),
        out_shape=jax.ShapeDtypeStruct((NPAD, D), _f32),
    )(h, w)


def _scale1_body(degp_ref, u_ref, mm_ref, dinv_ref):
    deg = degp_ref[0] + degp_ref[1] + 1.0
    dinv = jnp.where(deg > 0.0, lax.rsqrt(deg), 0.0)
    dinv_ref[...] = dinv
    mm_ref[...] = u_ref[...] * dinv


def _tc_scale1(deg_p, u1):
    return pl.pallas_call(
        _scale1_body,
        grid=(GRID,),
        in_specs=[
            pl.BlockSpec((NC, BN, D), lambda i: (0, i, 0)),
            pl.BlockSpec((BN, D), lambda i: (i, 0)),
        ],
        out_specs=[
            pl.BlockSpec((BN, D), lambda i: (i, 0)),
            pl.BlockSpec((BN, D), lambda i: (i, 0)),
        ],
        out_shape=[
            jax.ShapeDtypeStruct((NPAD, D), _f32),
            jax.ShapeDtypeStruct((NPAD, D), _f32),
        ],
    )(deg_p, u1)


def _layer_body(has_skip, *refs):
    if has_skip:
        a_ref, mm_ref, dinv_ref, b_ref, w_ref, skip_ref, h_ref, mmn_ref = refs
    else:
        a_ref, mm_ref, dinv_ref, b_ref, w_ref, h_ref, mmn_ref = refs
    dinv = dinv_ref[...]
    t = (a_ref[0] + a_ref[1] + mm_ref[...]) * dinv + b_ref[...]
    h = jnp.maximum(t, 0.0)
    if has_skip:
        h = h + skip_ref[...]
    h_ref[...] = h
    mmn_ref[...] = jnp.dot(h, w_ref[...], preferred_element_type=_f32) * dinv


def _tc_layer(a_p, mm, dinv16, b_row, w_next, skip=None):
    has_skip = skip is not None
    in_specs = [
        pl.BlockSpec((NC, BN, D), lambda i: (0, i, 0)),
        pl.BlockSpec((BN, D), lambda i: (i, 0)),
        pl.BlockSpec((BN, D), lambda i: (i, 0)),
        pl.BlockSpec((1, D), lambda i: (0, 0)),
        pl.BlockSpec((D, D), lambda i: (0, 0)),
    ]
    args = [a_p, mm, dinv16, b_row, w_next]
    if has_skip:
        in_specs.append(pl.BlockSpec((BN, D), lambda i: (i, 0)))
        args.append(skip)
    return pl.pallas_call(
        functools.partial(_layer_body, has_skip),
        grid=(GRID,),
        in_specs=in_specs,
        out_specs=[
            pl.BlockSpec((BN, D), lambda i: (i, 0)),
            pl.BlockSpec((BN, D), lambda i: (i, 0)),
        ],
        out_shape=[
            jax.ShapeDtypeStruct((NPAD, D), _f32),
            jax.ShapeDtypeStruct((NPAD, D), _f32),
        ],
    )(*args)


def _final_body(a_ref, mm_ref, dinv_ref, b_ref, x_ref, o_ref):
    t = (a_ref[0] + a_ref[1] + mm_ref[...]) * dinv_ref[...] + b_ref[...]
    o_ref[...] = jnp.maximum(t, 0.0) + x_ref[...]


def _tc_final(a_p, mm, dinv16, b_row, xp):
    return pl.pallas_call(
        _final_body,
        grid=(GRID,),
        in_specs=[
            pl.BlockSpec((NC, BN, D), lambda i: (0, i, 0)),
            pl.BlockSpec((BN, D), lambda i: (i, 0)),
            pl.BlockSpec((BN, D), lambda i: (i, 0)),
            pl.BlockSpec((1, D), lambda i: (0, 0)),
            pl.BlockSpec((BN, D), lambda i: (i, 0)),
        ],
        out_specs=pl.BlockSpec((BN, D), lambda i: (i, 0)),
        out_shape=jax.ShapeDtypeStruct((NPAD, D), _f32),
    )(a_p, mm, dinv16, b_row, xp)


def kernel(x, edge_index, W1, b1, W2, b2, W3, b3, W4, b4):
    ei = edge_index.astype(jnp.int32)
    pad = jnp.full((EPAD - E,), N, dtype=jnp.int32)
    srcc = jnp.concatenate([ei[0], pad]).reshape(TOTCH, CHUNK)
    dstc = jnp.concatenate([ei[1], pad]).reshape(TOTCH, CHUNK)
    dstp = dstc.reshape(NW, CPW, CHUNK)
    sdp = jnp.stack([srcc, dstc], axis=1)

    xp = jnp.pad(x.astype(_f32), ((0, NPAD - N), (0, 0)))
    zeros128 = jnp.zeros((NPAD, D), _f32)
    ones128 = jnp.ones((CHUNK, D), _f32)
    b1r, b2r, b3r, b4r = (b.reshape(1, D) for b in (b1, b2, b3, b4))

    deg_p = _sc_degree(dstp, zeros128, ones128)
    u1 = _tc_matmul(xp, W1)
    mm1, dinv_b = _tc_scale1(deg_p, u1)

    a1 = _sc_aggregate(mm1, sdp, zeros128)
    h1, mm2 = _tc_layer(a1, mm1, dinv_b, b1r, W2)
    a2 = _sc_aggregate(mm2, sdp, zeros128)
    _, mm3 = _tc_layer(a2, mm2, dinv_b, b2r, W3)
    a3 = _sc_aggregate(mm3, sdp, zeros128)
    _, mm4 = _tc_layer(a3, mm3, dinv_b, b3r, W4, skip=h1)
    a4 = _sc_aggregate(mm4, sdp, zeros128)
    out = _tc_final(a4, mm4, dinv_b, b4r, xp)
    return out[:N]


# speedup vs baseline: 1.3380x; 1.0001x over previous
import functools

import jax
import jax.numpy as jnp
from jax import lax
from jax.experimental import pallas as pl
from jax.experimental.pallas import tpu as pltpu
from jax.experimental.pallas import tpu_sc as plsc

N = 10000
D = 128
E = 320000

NC = 2
NS = 16
NW = NC * NS
CHUNK = 128
CPW = 80
EPAD = NW * CPW * CHUNK
NPAD = 10240
RPT = NPAD // NS

BN = 512
GRID = NPAD // BN

KS = 110
KF = 50
NKS0 = NS * KS
TOTCH = NW * CPW

_f32 = jnp.float32


def _mesh():
    return plsc.VectorSubcoreMesh(core_axis_name="c", subcore_axis_name="s")


def _deg_body(dst_hbm, zeros_hbm, ones_hbm, out_hbm, idx_v, ones_v, acc_sh, ssem):
    c = lax.axis_index("c")
    s = lax.axis_index("s")
    wid = c * NS + s
    r0 = s * RPT
    pltpu.sync_copy(zeros_hbm.at[pl.ds(r0, RPT)], acc_sh.at[pl.ds(r0, RPT)])
    pltpu.sync_copy(ones_hbm, ones_v)
    pltpu.sync_copy(dst_hbm.at[wid], idx_v)
    plsc.subcore_barrier()

    def fire(j, carry):
        pltpu.async_copy(ones_v, acc_sh.at[idx_v.at[j]], ssem, add=True)
        return carry

    lax.fori_loop(0, CPW, fire, 0)

    def drain(j, carry):
        pltpu.make_async_copy(ones_v, acc_sh.at[idx_v.at[0]], ssem).wait()
        return carry

    lax.fori_loop(0, CPW, drain, 0)
    plsc.subcore_barrier()
    pltpu.sync_copy(acc_sh.at[pl.ds(r0, RPT)], out_hbm.at[c, pl.ds(r0, RPT)])


def _sc_degree(dstp, zeros128, ones128):
    return pl.kernel(
        _deg_body,
        out_type=jax.ShapeDtypeStruct((NC, NPAD, D), _f32),
        mesh=_mesh(),
        scratch_types=[
            pltpu.VMEM((CPW, CHUNK), jnp.int32),
            pltpu.VMEM((CHUNK, D), _f32),
            pltpu.VMEM_SHARED((NPAD, D), _f32),
            pltpu.SemaphoreType.DMA,
        ],
    )(dstp, zeros128, ones128)


def _agg_body(table_hbm, sd_hbm, zeros_hbm, out_hbm,
              sd_r, rows_v, acc_sh, isem0, isem1, gsem0, gsem1):
    c = lax.axis_index("c")
    s = lax.axis_index("s")
    r0 = s * RPT
    base = jnp.where(c == 0, s * KS, NKS0 + s * KF)
    nch = jnp.where(c == 0, KS, KF)
    isems = (isem0, isem1)
    gsems = (gsem0, gsem1)
    pltpu.sync_copy(zeros_hbm.at[pl.ds(r0, RPT)], acc_sh.at[pl.ds(r0, RPT)])

    pltpu.async_copy(sd_hbm.at[base], sd_r.at[0], isems[0])
    pltpu.async_copy(sd_hbm.at[base + 1], sd_r.at[1], isems[1])
    plsc.subcore_barrier()
    pltpu.make_async_copy(sd_hbm.at[base], sd_r.at[0], isems[0]).wait()
    pltpu.async_copy(table_hbm.at[sd_r.at[0, 0]], rows_v.at[0], gsems[0])

    def group(g, carry):
        for b in range(2):
            j = 2 * g + b
            nb = 1 - b

            @pl.when(j + 1 < nch)
            def _fire_gather():
                pltpu.make_async_copy(
                    sd_hbm.at[base], sd_r.at[nb], isems[nb]).wait()
                pltpu.async_copy(
                    table_hbm.at[sd_r.at[nb, 0]], rows_v.at[nb], gsems[nb])

            pltpu.make_async_copy(
                table_hbm.at[sd_r.at[b, 0]], rows_v.at[b], gsems[b]).wait()
            pltpu.sync_copy(rows_v.at[b], acc_sh.at[sd_r.at[b, 1]], add=True)

            @pl.when(j + 2 < nch)
            def _fire_idx():
                pltpu.async_copy(
                    sd_hbm.at[base + (j + 2)], sd_r.at[b], isems[b])
        return carry

    lax.fori_loop(0, lax.div(nch, 2), group, 0)
    plsc.subcore_barrier()
    pltpu.sync_copy(acc_sh.at[pl.ds(r0, RPT)], out_hbm.at[c, pl.ds(r0, RPT)])


def _sc_aggregate(table, sd_flat, zeros128):
    return pl.kernel(
        _agg_body,
        out_type=jax.ShapeDtypeStruct((NC, NPAD, D), _f32),
        mesh=_mesh(),
        scratch_types=[
            pltpu.VMEM((2, 2, CHUNK), jnp.int32),
            pltpu.VMEM((2, CHUNK, D), _f32),
            pltpu.VMEM_SHARED((NPAD, D), _f32),
            pltpu.SemaphoreType.DMA,
            pltpu.SemaphoreType.DMA,
            pltpu.SemaphoreType.DMA,
            pltpu.SemaphoreType.DMA,
        ],
    )(table, sd_flat, zeros128)


def _mm_body(h_ref, w_ref, o_ref):
    o_ref[...] = jnp.dot(h_ref[...], w_ref[...], preferred_element_type=_f32)


def _tc_matmul(h, w):
    return pl.pallas_call(
        _mm_body,
        grid=(GRID,),
        in_specs=[
            pl.BlockSpec((BN, D), lambda i: (i, 0)),
            pl.BlockSpec((D, D), lambda i: (0, 0)),
        ],
        out_specs=pl.BlockSpec((BN, D), lambda i: (i, 0)),
        out_shape=jax.ShapeDtypeStruct((NPAD, D), _f32),
    )(h, w)


def _scale1_body(degp_ref, u_ref, mm_ref, dinv_ref):
    deg = degp_ref[0] + degp_ref[1] + 1.0
    dinv = jnp.where(deg > 0.0, lax.rsqrt(deg), 0.0)
    dinv_ref[...] = dinv
    mm_ref[...] = u_ref[...] * dinv


def _tc_scale1(deg_p, u1):
    return pl.pallas_call(
        _scale1_body,
        grid=(GRID,),
        in_specs=[
            pl.BlockSpec((NC, BN, D), lambda i: (0, i, 0)),
            pl.BlockSpec((BN, D), lambda i: (i, 0)),
        ],
        out_specs=[
            pl.BlockSpec((BN, D), lambda i: (i, 0)),
            pl.BlockSpec((BN, D), lambda i: (i, 0)),
        ],
        out_shape=[
            jax.ShapeDtypeStruct((NPAD, D), _f32),
            jax.ShapeDtypeStruct((NPAD, D), _f32),
        ],
    )(deg_p, u1)


def _layer_body(has_skip, *refs):
    if has_skip:
        a_ref, mm_ref, dinv_ref, b_ref, w_ref, skip_ref, h_ref, mmn_ref = refs
    else:
        a_ref, mm_ref, dinv_ref, b_ref, w_ref, h_ref, mmn_ref = refs
    dinv = dinv_ref[...]
    t = (a_ref[0] + a_ref[1] + mm_ref[...]) * dinv + b_ref[...]
    h = jnp.maximum(t, 0.0)
    if has_skip:
        h = h + skip_ref[...]
    h_ref[...] = h
    mmn_ref[...] = jnp.dot(h, w_ref[...], preferred_element_type=_f32) * dinv


def _tc_layer(a_p, mm, dinv16, b_row, w_next, skip=None):
    has_skip = skip is not None
    in_specs = [
        pl.BlockSpec((NC, BN, D), lambda i: (0, i, 0)),
        pl.BlockSpec((BN, D), lambda i: (i, 0)),
        pl.BlockSpec((BN, D), lambda i: (i, 0)),
        pl.BlockSpec((1, D), lambda i: (0, 0)),
        pl.BlockSpec((D, D), lambda i: (0, 0)),
    ]
    args = [a_p, mm, dinv16, b_row, w_next]
    if has_skip:
        in_specs.append(pl.BlockSpec((BN, D), lambda i: (i, 0)))
        args.append(skip)
    return pl.pallas_call(
        functools.partial(_layer_body, has_skip),
        grid=(GRID,),
        in_specs=in_specs,
        out_specs=[
            pl.BlockSpec((BN, D), lambda i: (i, 0)),
            pl.BlockSpec((BN, D), lambda i: (i, 0)),
        ],
        out_shape=[
            jax.ShapeDtypeStruct((NPAD, D), _f32),
            jax.ShapeDtypeStruct((NPAD, D), _f32),
        ],
    )(*args)


def _final_body(a_ref, mm_ref, dinv_ref, b_ref, x_ref, o_ref):
    t = (a_ref[0] + a_ref[1] + mm_ref[...]) * dinv_ref[...] + b_ref[...]
    o_ref[...] = jnp.maximum(t, 0.0) + x_ref[...]


def _tc_final(a_p, mm, dinv16, b_row, xp):
    return pl.pallas_call(
        _final_body,
        grid=(GRID,),
        in_specs=[
            pl.BlockSpec((NC, BN, D), lambda i: (0, i, 0)),
            pl.BlockSpec((BN, D), lambda i: (i, 0)),
            pl.BlockSpec((BN, D), lambda i: (i, 0)),
            pl.BlockSpec((1, D), lambda i: (0, 0)),
            pl.BlockSpec((BN, D), lambda i: (i, 0)),
        ],
        out_specs=pl.BlockSpec((BN, D), lambda i: (i, 0)),
        out_shape=jax.ShapeDtypeStruct((NPAD, D), _f32),
    )(a_p, mm, dinv16, b_row, xp)


def kernel(x, edge_index, W1, b1, W2, b2, W3, b3, W4, b4):
    ei = edge_index.astype(jnp.int32)
    pad = jnp.full((EPAD - E,), N, dtype=jnp.int32)
    srcc = jnp.concatenate([ei[0], pad]).reshape(TOTCH, CHUNK)
    dstc = jnp.concatenate([ei[1], pad]).reshape(TOTCH, CHUNK)
    dstp = dstc.reshape(NW, CPW, CHUNK)
    sdp = jnp.stack([srcc, dstc], axis=1)

    xp = jnp.pad(x.astype(_f32), ((0, NPAD - N), (0, 0)))
    zeros128 = jnp.zeros((NPAD, D), _f32)
    ones128 = jnp.ones((CHUNK, D), _f32)
    b1r, b2r, b3r, b4r = (b.reshape(1, D) for b in (b1, b2, b3, b4))

    deg_p = _sc_degree(dstp, zeros128, ones128)
    u1 = _tc_matmul(xp, W1)
    mm1, dinv_b = _tc_scale1(deg_p, u1)

    a1 = _sc_aggregate(mm1, sdp, zeros128)
    h1, mm2 = _tc_layer(a1, mm1, dinv_b, b1r, W2)
    a2 = _sc_aggregate(mm2, sdp, zeros128)
    _, mm3 = _tc_layer(a2, mm2, dinv_b, b2r, W3)
    a3 = _sc_aggregate(mm3, sdp, zeros128)
    _, mm4 = _tc_layer(a3, mm3, dinv_b, b3r, W4, skip=h1)
    a4 = _sc_aggregate(mm4, sdp, zeros128)
    out = _tc_final(a4, mm4, dinv_b, b4r, xp)
    return out[:N]
